# baseline (speedup 1.0000x reference)
import functools

import jax
import jax.numpy as jnp
from jax import lax
from jax.experimental import pallas as pl
from jax.experimental.pallas import tpu as pltpu
from jax.experimental.pallas import tpu_sc as plsc

_NC = 2
_NS = 16
_NW = _NC * _NS
_L = 16

_B, _SEQ, _D = 4096, 200, 64
_VOCAB = 5000

_SGRP = 8
_DGRP = _NW // _SGRP
_SPW = _SEQ // _SGRP
_DPW = _D // _DGRP
_NBT = _B // 128
_BTC = 8
_NCH = _NBT // _BTC
_NIT = _SPW * _NCH


def _prep_body(ts_ref, scale_ref, mm_ref):
    t = ts_ref[...]
    tmin = jnp.min(t)
    trange = jnp.max(t) - tmin
    safe = jnp.where(trange > 0, trange, jnp.float32(1.0))
    row = lax.broadcasted_iota(jnp.int32, (8, 128), 0)
    mm_ref[...] = jnp.where(
        row == 0, tmin, jnp.where(row == 1, safe, scale_ref[...])
    )


_prep = pl.pallas_call(
    _prep_body,
    out_shape=jax.ShapeDtypeStruct((8, 128), jnp.float32),
)


def _sc_body(
    x_hbm, ts_hbm, tbl_hbm, mm_hbm, out_hbm,
    mm_v, tbl_v, ts_v, x_v,
    sem_ts0, sem_ts1, sem_x0, sem_x1, sem_o0, sem_o1,
):
    wid = lax.axis_index("s") * _NC + lax.axis_index("c")
    si = wid // _DGRP
    d0 = (wid % _DGRP) * _DPW
    dt0 = (wid % _DGRP) * 2
    s_base = si * _SPW
    sem_ts = (sem_ts0, sem_ts1)
    sem_x = (sem_x0, sem_x1)
    sem_o = (sem_o0, sem_o1)

    pltpu.sync_copy(mm_hbm, mm_v)
    tmin = mm_v[0, pl.ds(0, _L)]
    tsafe = mm_v[1, pl.ds(0, _L)]
    tscale = mm_v[2, pl.ds(0, _L)]
    pltpu.sync_copy(tbl_hbm.at[:, pl.ds(d0, _DPW)], tbl_v)
    lane_iota = lax.iota(jnp.int32, _L)

    @plsc.parallel_loop(0, _VOCAB, unroll=4)
    def _swz(v):
        row = tbl_v[v, pl.ds(0, _L)]
        perm = (lane_iota + v) & (_L - 1)
        plsc.store_scatter(tbl_v, [jnp.zeros((_L,), jnp.int32) + v, perm], row)

    def coords(ch):
        s = s_base + ch // _NCH
        bt0 = (ch % _NCH) * _BTC
        return s, pl.multiple_of(bt0, _BTC)

    def start_loads(p, ch):
        s, bt0 = coords(ch)
        pltpu.async_copy(
            ts_hbm.at[s // 8, pl.ds(bt0, _BTC)], ts_v.at[p], sem_ts[p]
        )
        for j in range(2):
            pltpu.async_copy(
                x_hbm.at[s, dt0 + j, pl.ds(bt0, _BTC)], x_v.at[p, j], sem_x[p]
            )

    def wait_writeback(p):
        for j in range(2):
            pltpu.make_async_copy(
                x_v.at[p, j], out_hbm.at[0, j, pl.ds(0, _BTC)], sem_o[p]
            ).wait()

    dls = [jnp.full((_L,), d, jnp.int32) for d in range(_DPW)]

    def process(p, ch):
        s, _ = coords(ch)
        sl = s % 8
        pltpu.make_async_copy(
            ts_hbm.at[0, pl.ds(0, _BTC)], ts_v.at[p], sem_ts[p]
        ).wait()
        for j in range(2):
            pltpu.make_async_copy(
                x_hbm.at[0, 0, pl.ds(0, _BTC)], x_v.at[p, j], sem_x[p]
            ).wait()

        @plsc.parallel_loop(0, 128, step=_L, unroll=4)
        def _lanes(o):
            for bt in range(_BTC):
                t = ts_v[p, bt, sl, pl.ds(o, _L)]
                idx = ((t - tmin) / tsafe * jnp.float32(4999.0)).astype(jnp.int32)
                for j in range(2):
                    for di in range(8):
                        col = (idx + dls[j * 8 + di]) & (_L - 1)
                        g = plsc.load_gather(tbl_v, [idx, col])
                        x_v[p, j, bt, di, pl.ds(o, _L)] = (
                            x_v[p, j, bt, di, pl.ds(o, _L)] + tscale * g
                        )

    def start_writeback(p, ch):
        s, bt0 = coords(ch)
        for j in range(2):
            pltpu.async_copy(
                x_v.at[p, j], out_hbm.at[s, dt0 + j, pl.ds(bt0, _BTC)], sem_o[p]
            )

    def body(i, carry):
        ch0 = 2 * i
        ch1 = 2 * i + 1

        @pl.when(i > 0)
        def _():
            wait_writeback(0)

        start_loads(0, ch0)

        @pl.when(i > 0)
        def _():
            wait_writeback(1)

        start_loads(1, ch1)
        process(0, ch0)
        start_writeback(0, ch0)
        process(1, ch1)
        start_writeback(1, ch1)
        return carry

    lax.fori_loop(0, _NIT // 2, body, 0)
    wait_writeback(0)
    wait_writeback(1)


_sc = functools.partial(
    pl.kernel,
    out_type=jax.ShapeDtypeStruct((_SEQ, _D // 8, _NBT, 8, 128), jnp.float32),
    mesh=plsc.VectorSubcoreMesh(core_axis_name="c", subcore_axis_name="s"),
    scratch_types=[
        pltpu.VMEM((8, 128), jnp.float32),
        pltpu.VMEM((_VOCAB, _DPW), jnp.float32),
        pltpu.VMEM((2, _BTC, 8, 128), jnp.float32),
        pltpu.VMEM((2, 2, _BTC, 8, 128), jnp.float32),
    ] + [pltpu.SemaphoreType.DMA] * 6,
    compiler_params=pltpu.CompilerParams(
        use_tc_tiling_on_sc=False, needs_layout_passes=False
    ),
)(_sc_body)


def kernel(x, timestamps, pos_embedding, time_scale):
    x5 = (
        jnp.transpose(x, (1, 2, 0))
        .reshape(_SEQ, _D // 8, 8, _NBT, 128)
        .transpose(0, 1, 3, 2, 4)
    )
    ts4 = (
        jnp.transpose(timestamps, (1, 0))
        .reshape(_SEQ // 8, 8, _NBT, 128)
        .transpose(0, 2, 1, 3)
    )
    mm = _prep(ts4, time_scale.reshape(1, 1).astype(jnp.float32))
    out5 = _sc(x5, ts4, pos_embedding, mm)
    return (
        out5.transpose(0, 1, 3, 2, 4)
        .reshape(_SEQ, _D, _B)
        .transpose(2, 0, 1)
    )


# speedup vs baseline: 1.4763x; 1.4763x over previous
import functools

import jax
import jax.numpy as jnp
from jax import lax
from jax.experimental import pallas as pl
from jax.experimental.pallas import tpu as pltpu
from jax.experimental.pallas import tpu_sc as plsc

_NC = 2
_NS = 16
_NW = _NC * _NS
_L = 16

_B, _SEQ, _D = 4096, 200, 64
_VOCAB = 5000

_SGRP = 8
_DGRP = _NW // _SGRP
_SPW = _SEQ // _SGRP
_DPW = _D // _DGRP
_NBT = _B // 128
_BTC = 4
_NCH = _NBT // _BTC
_NIT = _SPW * _NCH


def _prep_body(ts_ref, scale_ref, mm_ref):
    t = ts_ref[...]
    tmin = jnp.min(t)
    trange = jnp.max(t) - tmin
    safe = jnp.where(trange > 0, trange, jnp.float32(1.0))
    row = lax.broadcasted_iota(jnp.int32, (8, 128), 0)
    mm_ref[...] = jnp.where(
        row == 0, tmin, jnp.where(row == 1, safe, scale_ref[...])
    )


_prep = pl.pallas_call(
    _prep_body,
    out_shape=jax.ShapeDtypeStruct((8, 128), jnp.float32),
)


def _sc_body(
    x_hbm, ts_hbm, tbl_hbm, mm_hbm, out_hbm,
    mm_v, tbl_v, ts_v, x_v,
    sem_ts0, sem_ts1, sem_x0, sem_x1, sem_o0, sem_o1,
):
    wid = lax.axis_index("s") * _NC + lax.axis_index("c")
    si = wid // _DGRP
    d0 = (wid % _DGRP) * _DPW
    dt0 = (wid % _DGRP) * 2
    s_base = si * _SPW
    sem_ts = (sem_ts0, sem_ts1)
    sem_x = (sem_x0, sem_x1)
    sem_o = (sem_o0, sem_o1)

    pltpu.sync_copy(mm_hbm, mm_v)
    tmin = mm_v[0, pl.ds(0, _L)]
    tsafe = mm_v[1, pl.ds(0, _L)]
    tscale = mm_v[2, pl.ds(0, _L)]
    pltpu.sync_copy(tbl_hbm.at[:, pl.ds(d0, _DPW)], tbl_v)
    lane_iota = lax.iota(jnp.int32, _L)

    @plsc.parallel_loop(0, _VOCAB, unroll=4)
    def _swz(v):
        row = tbl_v[v, pl.ds(0, _L)]
        perm = (lane_iota + v) & (_L - 1)
        plsc.store_scatter(tbl_v, [jnp.zeros((_L,), jnp.int32) + v, perm], row)

    def coords(ch):
        s = s_base + ch // _NCH
        bt0 = (ch % _NCH) * _BTC
        return s, pl.multiple_of(bt0, _BTC)

    def start_loads(p, ch):
        s, bt0 = coords(ch)
        pltpu.async_copy(
            ts_hbm.at[s // 8, pl.ds(bt0, _BTC)], ts_v.at[p], sem_ts[p]
        )
        for j in range(2):
            pltpu.async_copy(
                x_hbm.at[s, dt0 + j, pl.ds(bt0, _BTC)], x_v.at[p, j], sem_x[p]
            )

    def wait_writeback(p):
        for j in range(2):
            pltpu.make_async_copy(
                x_v.at[p, j], out_hbm.at[0, j, pl.ds(0, _BTC)], sem_o[p]
            ).wait()

    dls = [jnp.full((_L,), d, jnp.int32) for d in range(_DPW)]

    def process(p, ch):
        s, _ = coords(ch)
        sl = s % 8
        pltpu.make_async_copy(
            ts_hbm.at[0, pl.ds(0, _BTC)], ts_v.at[p], sem_ts[p]
        ).wait()
        for j in range(2):
            pltpu.make_async_copy(
                x_hbm.at[0, 0, pl.ds(0, _BTC)], x_v.at[p, j], sem_x[p]
            ).wait()

        @plsc.parallel_loop(0, 128, step=_L, unroll=4)
        def _lanes(o):
            for bt in range(_BTC):
                t = ts_v[p, bt, sl, pl.ds(o, _L)]
                idx = ((t - tmin) / tsafe * jnp.float32(4999.0)).astype(jnp.int32)
                for j in range(2):
                    for di in range(8):
                        col = (idx + dls[j * 8 + di]) & (_L - 1)
                        g = plsc.load_gather(tbl_v, [idx, col])
                        x_v[p, j, bt, di, pl.ds(o, _L)] = (
                            x_v[p, j, bt, di, pl.ds(o, _L)] + tscale * g
                        )

    def start_writeback(p, ch):
        s, bt0 = coords(ch)
        for j in range(2):
            pltpu.async_copy(
                x_v.at[p, j], out_hbm.at[s, dt0 + j, pl.ds(bt0, _BTC)], sem_o[p]
            )

    def body(i, carry):
        ch0 = 2 * i
        ch1 = 2 * i + 1

        @pl.when(i > 0)
        def _():
            wait_writeback(0)

        start_loads(0, ch0)

        @pl.when(i > 0)
        def _():
            wait_writeback(1)

        start_loads(1, ch1)
        process(0, ch0)
        start_writeback(0, ch0)
        process(1, ch1)
        start_writeback(1, ch1)
        return carry

    lax.fori_loop(0, _NIT // 2, body, 0)
    wait_writeback(0)
    wait_writeback(1)


_sc = functools.partial(
    pl.kernel,
    out_type=jax.ShapeDtypeStruct((_SEQ, _D // 8, _NBT, 8, 128), jnp.float32),
    mesh=plsc.VectorSubcoreMesh(core_axis_name="c", subcore_axis_name="s"),
    scratch_types=[
        pltpu.VMEM((8, 128), jnp.float32),
        pltpu.VMEM((_VOCAB, _DPW), jnp.float32),
        pltpu.VMEM((2, _BTC, 8, 128), jnp.float32),
        pltpu.VMEM((2, 2, _BTC, 8, 128), jnp.float32),
    ] + [pltpu.SemaphoreType.DMA] * 6,
    compiler_params=pltpu.CompilerParams(
        use_tc_tiling_on_sc=False, needs_layout_passes=False
    ),
)(_sc_body)


def kernel(x, timestamps, pos_embedding, time_scale):
    x5 = (
        jnp.transpose(x, (1, 2, 0))
        .reshape(_SEQ, _D // 8, 8, _NBT, 128)
        .transpose(0, 1, 3, 2, 4)
    )
    ts4 = (
        jnp.transpose(timestamps, (1, 0))
        .reshape(_SEQ // 8, 8, _NBT, 128)
        .transpose(0, 2, 1, 3)
    )
    mm = _prep(ts4, time_scale.reshape(1, 1).astype(jnp.float32))
    out5 = _sc(x5, ts4, pos_embedding, mm)
    return (
        out5.transpose(0, 1, 3, 2, 4)
        .reshape(_SEQ, _D, _B)
        .transpose(2, 0, 1)
    )


# speedup vs baseline: 2.6058x; 1.7651x over previous
import functools

import jax
import jax.numpy as jnp
from jax import lax
from jax.experimental import pallas as pl
from jax.experimental.pallas import tpu as pltpu
from jax.experimental.pallas import tpu_sc as plsc

_NC = 2
_NS = 16
_NW = _NC * _NS
_L = 16

_B, _SEQ, _D = 4096, 200, 64
_VOCAB = 5000

_SGRP = 8
_DGRP = _NW // _SGRP
_SPW = _SEQ // _SGRP
_DPW = _D // _DGRP
_NBT = _B // 128
_BTC = 4
_NCH = _NBT // _BTC
_NIT = _SPW * _NCH


def _prep_body(ts_ref, scale_ref, mm_ref):
    t = ts_ref[...]
    tmin = jnp.min(t)
    trange = jnp.max(t) - tmin
    safe = jnp.where(trange > 0, trange, jnp.float32(1.0))
    row = lax.broadcasted_iota(jnp.int32, (8, 128), 0)
    mm_ref[...] = jnp.where(
        row == 0, tmin, jnp.where(row == 1, safe, scale_ref[...])
    )


_prep = pl.pallas_call(
    _prep_body,
    out_shape=jax.ShapeDtypeStruct((8, 128), jnp.float32),
)


def _sc_body(
    x_hbm, ts_hbm, tbl_hbm, mm_hbm, out_hbm,
    mm_v, tbl_v, ts_v, x_v,
    sem_ts0, sem_ts1, sem_x0, sem_x1, sem_o0, sem_o1,
):
    wid = lax.axis_index("s") * _NC + lax.axis_index("c")
    si = wid // _DGRP
    d0 = (wid % _DGRP) * _DPW
    dt0 = (wid % _DGRP) * 2
    s_base = si * _SPW
    sem_ts = (sem_ts0, sem_ts1)
    sem_x = (sem_x0, sem_x1)
    sem_o = (sem_o0, sem_o1)

    pltpu.sync_copy(mm_hbm, mm_v)
    tmin = mm_v[0, pl.ds(0, _L)]
    tsafe = mm_v[1, pl.ds(0, _L)]
    tscale = mm_v[2, pl.ds(0, _L)]
    pltpu.sync_copy(tbl_hbm.at[:, pl.ds(d0, _DPW)], tbl_v)
    lane_iota = lax.iota(jnp.int32, _L)

    @plsc.parallel_loop(0, _VOCAB, unroll=4)
    def _swz(v):
        row = tbl_v[v, pl.ds(0, _L)]
        perm = (lane_iota + v) & (_L - 1)
        plsc.store_scatter(tbl_v, [jnp.zeros((_L,), jnp.int32) + v, perm], row)

    def coords(ch):
        s = s_base + ch // _NCH
        bt0 = (ch % _NCH) * _BTC
        return s, pl.multiple_of(bt0, _BTC)

    def start_loads(p, ch):
        s, bt0 = coords(ch)
        pltpu.async_copy(
            ts_hbm.at[s // 8, pl.ds(bt0, _BTC)], ts_v.at[p], sem_ts[p]
        )
        for j in range(2):
            pltpu.async_copy(
                x_hbm.at[s, dt0 + j, pl.ds(bt0, _BTC)], x_v.at[p, j], sem_x[p]
            )

    def wait_writeback(p):
        for j in range(2):
            pltpu.make_async_copy(
                x_v.at[p, j], out_hbm.at[0, j, pl.ds(0, _BTC)], sem_o[p]
            ).wait()

    dls = [jnp.full((_L,), d, jnp.int32) for d in range(_DPW)]

    def process(p, ch):
        s, _ = coords(ch)
        sl = s % 8
        pltpu.make_async_copy(
            ts_hbm.at[0, pl.ds(0, _BTC)], ts_v.at[p], sem_ts[p]
        ).wait()
        for j in range(2):
            pltpu.make_async_copy(
                x_hbm.at[0, 0, pl.ds(0, _BTC)], x_v.at[p, j], sem_x[p]
            ).wait()

        @plsc.parallel_loop(0, 128, step=_L, unroll=2)
        def _lanes(o):
            for bt in range(_BTC):
                t = ts_v[p, bt, sl, pl.ds(o, _L)]
                idx = ((t - tmin) / tsafe * jnp.float32(4999.0)).astype(jnp.int32)
                for j in range(2):
                    for di in range(8):
                        col = (idx + dls[j * 8 + di]) & (_L - 1)
                        g = plsc.load_gather(tbl_v, [idx, col])
                        x_v[p, j, bt, di, pl.ds(o, _L)] = (
                            x_v[p, j, bt, di, pl.ds(o, _L)] + tscale * g
                        )

    def start_writeback(p, ch):
        s, bt0 = coords(ch)
        for j in range(2):
            pltpu.async_copy(
                x_v.at[p, j], out_hbm.at[s, dt0 + j, pl.ds(bt0, _BTC)], sem_o[p]
            )

    def body(i, carry):
        ch0 = 2 * i
        ch1 = 2 * i + 1

        @pl.when(i > 0)
        def _():
            wait_writeback(0)

        start_loads(0, ch0)

        @pl.when(i > 0)
        def _():
            wait_writeback(1)

        start_loads(1, ch1)
        process(0, ch0)
        start_writeback(0, ch0)
        process(1, ch1)
        start_writeback(1, ch1)
        return carry

    lax.fori_loop(0, _NIT // 2, body, 0)
    wait_writeback(0)
    wait_writeback(1)


_sc = functools.partial(
    pl.kernel,
    out_type=jax.ShapeDtypeStruct((_SEQ, _D // 8, _NBT, 8, 128), jnp.float32),
    mesh=plsc.VectorSubcoreMesh(core_axis_name="c", subcore_axis_name="s"),
    scratch_types=[
        pltpu.VMEM((8, 128), jnp.float32),
        pltpu.VMEM((_VOCAB, _DPW), jnp.float32),
        pltpu.VMEM((2, _BTC, 8, 128), jnp.float32),
        pltpu.VMEM((2, 2, _BTC, 8, 128), jnp.float32),
    ] + [pltpu.SemaphoreType.DMA] * 6,
    compiler_params=pltpu.CompilerParams(
        use_tc_tiling_on_sc=False, needs_layout_passes=False
    ),
)(_sc_body)


def kernel(x, timestamps, pos_embedding, time_scale):
    x5 = (
        jnp.transpose(x, (1, 2, 0))
        .reshape(_SEQ, _D // 8, 8, _NBT, 128)
        .transpose(0, 1, 3, 2, 4)
    )
    ts4 = (
        jnp.transpose(timestamps, (1, 0))
        .reshape(_SEQ // 8, 8, _NBT, 128)
        .transpose(0, 2, 1, 3)
    )
    mm = _prep(ts4, time_scale.reshape(1, 1).astype(jnp.float32))
    out5 = _sc(x5, ts4, pos_embedding, mm)
    return (
        out5.transpose(0, 1, 3, 2, 4)
        .reshape(_SEQ, _D, _B)
        .transpose(2, 0, 1)
    )


# speedup vs baseline: 4.4080x; 1.6916x over previous
import functools

import jax
import jax.numpy as jnp
from jax import lax
from jax.experimental import pallas as pl
from jax.experimental.pallas import tpu as pltpu
from jax.experimental.pallas import tpu_sc as plsc

_NC = 2
_NS = 16
_NW = _NC * _NS
_L = 16

_B, _SEQ, _D = 4096, 200, 64
_VOCAB = 5000

_SGRP = 8
_DGRP = _NW // _SGRP
_SPW = _SEQ // _SGRP
_DPW = _D // _DGRP
_NBT = _B // 128
_BTC = 4
_NCH = _NBT // _BTC
_NIT = _SPW * _NCH


def _prep_body(ts_ref, scale_ref, mm_ref):
    t = ts_ref[...]
    tmin = jnp.min(t)
    trange = jnp.max(t) - tmin
    safe = jnp.where(trange > 0, trange, jnp.float32(1.0))
    row = lax.broadcasted_iota(jnp.int32, (8, 128), 0)
    mm_ref[...] = jnp.where(
        row == 0, tmin, jnp.where(row == 1, safe, scale_ref[...])
    )


_prep = pl.pallas_call(
    _prep_body,
    out_shape=jax.ShapeDtypeStruct((8, 128), jnp.float32),
)


def _sc_body(
    x_hbm, ts_hbm, tbl_hbm, mm_hbm, out_hbm,
    mm_v, tbl_v, ts_v, x_v,
    sem_ts0, sem_ts1, sem_x0, sem_x1, sem_o0, sem_o1,
):
    wid = lax.axis_index("s") * _NC + lax.axis_index("c")
    si = wid // _DGRP
    d0 = (wid % _DGRP) * _DPW
    dt0 = (wid % _DGRP) * 2
    s_base = si * _SPW
    sem_ts = (sem_ts0, sem_ts1)
    sem_x = (sem_x0, sem_x1)
    sem_o = (sem_o0, sem_o1)

    pltpu.sync_copy(mm_hbm, mm_v)
    tmin = mm_v[0, pl.ds(0, _L)]
    tsafe = mm_v[1, pl.ds(0, _L)]
    tscale = mm_v[2, pl.ds(0, _L)]
    pltpu.sync_copy(tbl_hbm.at[:, pl.ds(d0, _DPW)], tbl_v)
    lane_iota = lax.iota(jnp.int32, _L)

    @plsc.parallel_loop(0, _VOCAB, unroll=4)
    def _swz(v):
        row = tbl_v[v, pl.ds(0, _L)]
        perm = (lane_iota + v) & (_L - 1)
        plsc.store_scatter(tbl_v, [jnp.zeros((_L,), jnp.int32) + v, perm], row)

    def coords(ch):
        s = s_base + ch // _NCH
        bt0 = (ch % _NCH) * _BTC
        return s, pl.multiple_of(bt0, _BTC)

    def start_loads(p, ch):
        s, bt0 = coords(ch)
        pltpu.async_copy(
            ts_hbm.at[s // 8, pl.ds(bt0, _BTC)], ts_v.at[p], sem_ts[p]
        )
        for j in range(2):
            pltpu.async_copy(
                x_hbm.at[s, dt0 + j, pl.ds(bt0, _BTC)], x_v.at[p, j], sem_x[p]
            )

    def wait_writeback(p):
        for j in range(2):
            pltpu.make_async_copy(
                x_v.at[p, j], out_hbm.at[0, j, pl.ds(0, _BTC)], sem_o[p]
            ).wait()

    dls = [jnp.full((_L,), d, jnp.int32) for d in range(_DPW)]

    def process(p, ch):
        s, _ = coords(ch)
        sl = s % 8
        pltpu.make_async_copy(
            ts_hbm.at[0, pl.ds(0, _BTC)], ts_v.at[p], sem_ts[p]
        ).wait()
        for j in range(2):
            pltpu.make_async_copy(
                x_hbm.at[0, 0, pl.ds(0, _BTC)], x_v.at[p, j], sem_x[p]
            ).wait()

        @plsc.parallel_loop(0, 128, step=_L, unroll=2)
        def _lanes(o):
            for bt in range(_BTC):
                t = ts_v[p, bt, sl, pl.ds(o, _L)]
                idx = ((t - tmin) / tsafe * jnp.float32(4999.0)).astype(jnp.int32)
                acc = []
                for dl in range(_DPW):
                    col = (idx + dls[dl]) & (_L - 1)
                    acc.append(plsc.load_gather(tbl_v, [idx, col]))
                for dl in range(_DPW):
                    j, di = divmod(dl, 8)
                    acc[dl] = x_v[p, j, bt, di, pl.ds(o, _L)] + tscale * acc[dl]
                for dl in range(_DPW):
                    j, di = divmod(dl, 8)
                    x_v[p, j, bt, di, pl.ds(o, _L)] = acc[dl]

    def start_writeback(p, ch):
        s, bt0 = coords(ch)
        for j in range(2):
            pltpu.async_copy(
                x_v.at[p, j], out_hbm.at[s, dt0 + j, pl.ds(bt0, _BTC)], sem_o[p]
            )

    def body(i, carry):
        ch0 = 2 * i
        ch1 = 2 * i + 1

        @pl.when(i > 0)
        def _():
            wait_writeback(0)

        start_loads(0, ch0)

        @pl.when(i > 0)
        def _():
            wait_writeback(1)

        start_loads(1, ch1)
        process(0, ch0)
        start_writeback(0, ch0)
        process(1, ch1)
        start_writeback(1, ch1)
        return carry

    lax.fori_loop(0, _NIT // 2, body, 0)
    wait_writeback(0)
    wait_writeback(1)


_sc = functools.partial(
    pl.kernel,
    out_type=jax.ShapeDtypeStruct((_SEQ, _D // 8, _NBT, 8, 128), jnp.float32),
    mesh=plsc.VectorSubcoreMesh(core_axis_name="c", subcore_axis_name="s"),
    scratch_types=[
        pltpu.VMEM((8, 128), jnp.float32),
        pltpu.VMEM((_VOCAB, _DPW), jnp.float32),
        pltpu.VMEM((2, _BTC, 8, 128), jnp.float32),
        pltpu.VMEM((2, 2, _BTC, 8, 128), jnp.float32),
    ] + [pltpu.SemaphoreType.DMA] * 6,
    compiler_params=pltpu.CompilerParams(
        use_tc_tiling_on_sc=False, needs_layout_passes=False
    ),
)(_sc_body)


def kernel(x, timestamps, pos_embedding, time_scale):
    x5 = (
        jnp.transpose(x, (1, 2, 0))
        .reshape(_SEQ, _D // 8, 8, _NBT, 128)
        .transpose(0, 1, 3, 2, 4)
    )
    ts4 = (
        jnp.transpose(timestamps, (1, 0))
        .reshape(_SEQ // 8, 8, _NBT, 128)
        .transpose(0, 2, 1, 3)
    )
    mm = _prep(ts4, time_scale.reshape(1, 1).astype(jnp.float32))
    out5 = _sc(x5, ts4, pos_embedding, mm)
    return (
        out5.transpose(0, 1, 3, 2, 4)
        .reshape(_SEQ, _D, _B)
        .transpose(2, 0, 1)
    )


# speedup vs baseline: 4.4497x; 1.0094x over previous
import functools

import jax
import jax.numpy as jnp
from jax import lax
from jax.experimental import pallas as pl
from jax.experimental.pallas import tpu as pltpu
from jax.experimental.pallas import tpu_sc as plsc

_NC = 2
_NS = 16
_NW = _NC * _NS
_L = 16

_B, _SEQ, _D = 4096, 200, 64
_VOCAB = 5000

_SGRP = 8
_DGRP = _NW // _SGRP
_SPW = _SEQ // _SGRP
_DPW = _D // _DGRP
_NBT = _B // 128
_BTC = 8
_NCH = _NBT // _BTC
_NIT = _SPW * _NCH


def _prep_body(ts_ref, scale_ref, mm_ref):
    t = ts_ref[...]
    tmin = jnp.min(t)
    trange = jnp.max(t) - tmin
    safe = jnp.where(trange > 0, trange, jnp.float32(1.0))
    row = lax.broadcasted_iota(jnp.int32, (8, 128), 0)
    mm_ref[...] = jnp.where(
        row == 0, tmin, jnp.where(row == 1, safe, scale_ref[...])
    )


_prep = pl.pallas_call(
    _prep_body,
    out_shape=jax.ShapeDtypeStruct((8, 128), jnp.float32),
)


def _sc_body(
    x_hbm, ts_hbm, tbl_hbm, mm_hbm, out_hbm,
    mm_v, tbl_v, ts_v, x_v,
    sem_ts0, sem_ts1, sem_x0, sem_x1, sem_o0, sem_o1,
):
    wid = lax.axis_index("s") * _NC + lax.axis_index("c")
    si = wid // _DGRP
    d0 = (wid % _DGRP) * _DPW
    dt0 = (wid % _DGRP) * 2
    s_base = si * _SPW
    sem_ts = (sem_ts0, sem_ts1)
    sem_x = (sem_x0, sem_x1)
    sem_o = (sem_o0, sem_o1)

    pltpu.sync_copy(mm_hbm, mm_v)
    tmin = mm_v[0, pl.ds(0, _L)]
    tsafe = mm_v[1, pl.ds(0, _L)]
    tscale = mm_v[2, pl.ds(0, _L)]
    pltpu.sync_copy(tbl_hbm.at[:, pl.ds(d0, _DPW)], tbl_v)
    lane_iota = lax.iota(jnp.int32, _L)

    @plsc.parallel_loop(0, _VOCAB, unroll=4)
    def _swz(v):
        row = tbl_v[v, pl.ds(0, _L)]
        perm = (lane_iota + v) & (_L - 1)
        plsc.store_scatter(tbl_v, [jnp.zeros((_L,), jnp.int32) + v, perm], row)

    def coords(ch):
        s = s_base + ch // _NCH
        bt0 = (ch % _NCH) * _BTC
        return s, pl.multiple_of(bt0, _BTC)

    def start_loads(p, ch):
        s, bt0 = coords(ch)
        pltpu.async_copy(
            ts_hbm.at[s // 8, pl.ds(bt0, _BTC)], ts_v.at[p], sem_ts[p]
        )
        for j in range(2):
            pltpu.async_copy(
                x_hbm.at[s, dt0 + j, pl.ds(bt0, _BTC)], x_v.at[p, j], sem_x[p]
            )

    def wait_writeback(p):
        for j in range(2):
            pltpu.make_async_copy(
                x_v.at[p, j], out_hbm.at[0, j, pl.ds(0, _BTC)], sem_o[p]
            ).wait()

    dls = [jnp.full((_L,), d, jnp.int32) for d in range(_DPW)]

    def process(p, ch):
        s, _ = coords(ch)
        sl = s % 8
        pltpu.make_async_copy(
            ts_hbm.at[0, pl.ds(0, _BTC)], ts_v.at[p], sem_ts[p]
        ).wait()
        for j in range(2):
            pltpu.make_async_copy(
                x_hbm.at[0, 0, pl.ds(0, _BTC)], x_v.at[p, j], sem_x[p]
            ).wait()

        @plsc.parallel_loop(0, 128, step=_L, unroll=2)
        def _lanes(o):
            for bt in range(_BTC):
                t = ts_v[p, bt, sl, pl.ds(o, _L)]
                idx = ((t - tmin) / tsafe * jnp.float32(4999.0)).astype(jnp.int32)
                acc = []
                for dl in range(_DPW):
                    col = (idx + dls[dl]) & (_L - 1)
                    acc.append(plsc.load_gather(tbl_v, [idx, col]))
                for dl in range(_DPW):
                    j, di = divmod(dl, 8)
                    acc[dl] = x_v[p, j, bt, di, pl.ds(o, _L)] + tscale * acc[dl]
                for dl in range(_DPW):
                    j, di = divmod(dl, 8)
                    x_v[p, j, bt, di, pl.ds(o, _L)] = acc[dl]

    def start_writeback(p, ch):
        s, bt0 = coords(ch)
        for j in range(2):
            pltpu.async_copy(
                x_v.at[p, j], out_hbm.at[s, dt0 + j, pl.ds(bt0, _BTC)], sem_o[p]
            )

    def body(i, carry):
        ch0 = 2 * i
        ch1 = 2 * i + 1

        @pl.when(i > 0)
        def _():
            wait_writeback(0)

        start_loads(0, ch0)

        @pl.when(i > 0)
        def _():
            wait_writeback(1)

        start_loads(1, ch1)
        process(0, ch0)
        start_writeback(0, ch0)
        process(1, ch1)
        start_writeback(1, ch1)
        return carry

    lax.fori_loop(0, _NIT // 2, body, 0)
    wait_writeback(0)
    wait_writeback(1)


_sc = functools.partial(
    pl.kernel,
    out_type=jax.ShapeDtypeStruct((_SEQ, _D // 8, _NBT, 8, 128), jnp.float32),
    mesh=plsc.VectorSubcoreMesh(core_axis_name="c", subcore_axis_name="s"),
    scratch_types=[
        pltpu.VMEM((8, 128), jnp.float32),
        pltpu.VMEM((_VOCAB, _DPW), jnp.float32),
        pltpu.VMEM((2, _BTC, 8, 128), jnp.float32),
        pltpu.VMEM((2, 2, _BTC, 8, 128), jnp.float32),
    ] + [pltpu.SemaphoreType.DMA] * 6,
    compiler_params=pltpu.CompilerParams(
        use_tc_tiling_on_sc=False, needs_layout_passes=False
    ),
)(_sc_body)


def kernel(x, timestamps, pos_embedding, time_scale):
    x5 = (
        jnp.transpose(x, (1, 2, 0))
        .reshape(_SEQ, _D // 8, 8, _NBT, 128)
        .transpose(0, 1, 3, 2, 4)
    )
    ts4 = (
        jnp.transpose(timestamps, (1, 0))
        .reshape(_SEQ // 8, 8, _NBT, 128)
        .transpose(0, 2, 1, 3)
    )
    mm = _prep(ts4, time_scale.reshape(1, 1).astype(jnp.float32))
    out5 = _sc(x5, ts4, pos_embedding, mm)
    return (
        out5.transpose(0, 1, 3, 2, 4)
        .reshape(_SEQ, _D, _B)
        .transpose(2, 0, 1)
    )
